# speedup vs baseline: 1.9638x; 1.9638x over previous
import functools

import jax
import jax.numpy as jnp
from jax import lax
from jax.experimental import pallas as pl
from jax.experimental.pallas import tpu as pltpu
from jax.experimental.pallas import tpu_sc as plsc


_CH = 128


def _make_sc_gather(L_pad, D, n_chunks_per_worker, NC, NS):
    NW = NC * NS
    b_per_w = L_pad // NW
    mesh = plsc.VectorSubcoreMesh(core_axis_name="c", subcore_axis_name="s")

    @functools.partial(
        pl.kernel,
        mesh=mesh,
        out_type=jax.ShapeDtypeStruct((L_pad, D), jnp.float32),
        scratch_types=[
            pltpu.VMEM((_CH,), jnp.int32),
            pltpu.VMEM((_CH, D), jnp.float32),
            pltpu.SemaphoreType.DMA,
        ],
    )
    def gather_k(idx_hbm, table_hbm, out_hbm, idx_v, rows_v, sem):
        wid = lax.axis_index("s") * NC + lax.axis_index("c")
        base_w = wid * b_per_w

        def body(it, carry):
            base = base_w + it * _CH
            pltpu.sync_copy(idx_hbm.at[pl.ds(base, _CH)], idx_v)
            pltpu.async_copy(table_hbm.at[idx_v], rows_v, sem).wait()
            pltpu.sync_copy(rows_v, out_hbm.at[pl.ds(base, _CH)])
            return carry

        lax.fori_loop(0, n_chunks_per_worker, body, 0)

    return gather_k


def _mlp_body(x_ref, wi_ref, bi_ref, wo_ref, bo_ref, o_ref):
    x = x_ref[...]
    h = lax.dot_general(x, wi_ref[...], (((1,), (1,)), ((), ())),
                        preferred_element_type=jnp.float32) + bi_ref[...]
    h = h * jnp.tanh(jax.nn.softplus(h))
    o = lax.dot_general(h, wo_ref[...], (((1,), (1,)), ((), ())),
                        preferred_element_type=jnp.float32) + bo_ref[...]
    o_ref[...] = x + o


def _mlp_call(x, wi, bi, wo, bo, block_rows):
    R, F = x.shape
    grid = (R // block_rows,)
    return pl.pallas_call(
        _mlp_body,
        grid=grid,
        in_specs=[
            pl.BlockSpec((block_rows, F), lambda i: (i, 0)),
            pl.BlockSpec((F, F), lambda i: (0, 0)),
            pl.BlockSpec((1, F), lambda i: (0, 0)),
            pl.BlockSpec((F, F), lambda i: (0, 0)),
            pl.BlockSpec((1, F), lambda i: (0, 0)),
        ],
        out_specs=pl.BlockSpec((block_rows, F), lambda i: (i, 0)),
        out_shape=jax.ShapeDtypeStruct((R, F), jnp.float32),
    )(x, wi, bi.reshape(1, F), wo, bo.reshape(1, F))


def kernel(node_embeddings, atoms_edge, atoms_label,
           W_inner_edge, b_inner_edge, W_outer_edge, b_outer_edge,
           W_inner_label, b_inner_label, W_outer_label, b_outer_label):
    N, D = node_embeddings.shape
    E2 = atoms_edge.shape[0]
    NL = atoms_label.shape[0]
    L = E2 + NL

    info = plsc.get_sparse_core_info()
    NC, NS = info.num_cores, info.num_subcores
    NW = NC * NS
    align = NW * _CH
    n_chunks_total = -(-L // align)
    L_pad = n_chunks_total * align
    n_chunks_per_worker = L_pad // (NW * _CH)

    pad = L_pad - L
    idx_all = jnp.concatenate([
        atoms_edge, atoms_label,
        jnp.zeros((pad,), dtype=jnp.int32),
    ])

    gather_k = _make_sc_gather(L_pad, D, n_chunks_per_worker, NC, NS)
    gathered = gather_k(idx_all, node_embeddings)

    ge = gathered[:E2].reshape(E2 // 2, 2 * D)
    gl = gathered[E2:E2 + NL]

    msg_e = _mlp_call(ge, W_inner_edge, b_inner_edge, W_outer_edge,
                      b_outer_edge, block_rows=1280)
    msg_l = _mlp_call(gl, W_inner_label, b_inner_label, W_outer_label,
                      b_outer_label, block_rows=2000)

    output_messages = jnp.concatenate([msg_e.reshape(-1, D), msg_l], axis=0)
    output_indices = jnp.concatenate([atoms_edge, atoms_label], axis=0)
    return (output_messages, output_indices)


# speedup vs baseline: 3.9541x; 2.0134x over previous
import functools

import jax
import jax.numpy as jnp
from jax import lax
from jax.experimental import pallas as pl
from jax.experimental.pallas import tpu as pltpu
from jax.experimental.pallas import tpu_sc as plsc


_CH = 128


def _make_sc_gather(L_pad, D, n_chunks_per_worker, NC, NS):
    NW = NC * NS
    b_per_w = L_pad // NW
    mesh = plsc.VectorSubcoreMesh(core_axis_name="c", subcore_axis_name="s")

    @functools.partial(
        pl.kernel,
        mesh=mesh,
        out_type=jax.ShapeDtypeStruct((L_pad, D), jnp.float32),
        scratch_types=[
            pltpu.VMEM((_CH,), jnp.int32),
            pltpu.VMEM((_CH, D), jnp.float32),
            pltpu.SemaphoreType.DMA,
        ],
    )
    def gather_k(idx_hbm, table_hbm, out_hbm, idx_v, rows_v, sem):
        wid = lax.axis_index("s") * NC + lax.axis_index("c")
        base_w = wid * b_per_w

        def body(it, carry):
            base = base_w + it * _CH
            pltpu.sync_copy(idx_hbm.at[pl.ds(base, _CH)], idx_v)
            pltpu.async_copy(table_hbm.at[idx_v], rows_v, sem).wait()
            pltpu.sync_copy(rows_v, out_hbm.at[pl.ds(base, _CH)])
            return carry

        lax.fori_loop(0, n_chunks_per_worker, body, 0)

    return gather_k


_BR = 2000


def _mlp2(x, wi, bi, wo, bo):
    h = lax.dot_general(x, wi, (((1,), (1,)), ((), ())),
                        preferred_element_type=jnp.float32) + bi
    h = h * jnp.tanh(jax.nn.softplus(h))
    o = lax.dot_general(h, wo, (((1,), (1,)), ((), ())),
                        preferred_element_type=jnp.float32) + bo
    return x + o


def _make_mlp_body(n_edge_blocks, D):
    def body(x_ref, wie, bie, woe, boe, wil, bil, wol, bol, o_ref):
        pid = pl.program_id(0)

        @pl.when(pid < n_edge_blocks)
        def _():
            x = x_ref[...].reshape(_BR // 2, 2 * D)
            o = _mlp2(x, wie[...], bie[...], woe[...], boe[...])
            o_ref[...] = o.reshape(_BR, D)

        @pl.when(pid >= n_edge_blocks)
        def _():
            o_ref[...] = _mlp2(x_ref[...], wil[...], bil[...], wol[...],
                               bol[...])

    return body


def _mlp_call(gathered, L, n_edge_blocks, D,
              wie, bie, woe, boe, wil, bil, wol, bol):
    grid = (L // _BR,)
    full = lambda i: (0, 0)
    return pl.pallas_call(
        _make_mlp_body(n_edge_blocks, D),
        grid=grid,
        in_specs=[
            pl.BlockSpec((_BR, D), lambda i: (i, 0)),
            pl.BlockSpec((2 * D, 2 * D), full),
            pl.BlockSpec((1, 2 * D), full),
            pl.BlockSpec((2 * D, 2 * D), full),
            pl.BlockSpec((1, 2 * D), full),
            pl.BlockSpec((D, D), full),
            pl.BlockSpec((1, D), full),
            pl.BlockSpec((D, D), full),
            pl.BlockSpec((1, D), full),
        ],
        out_specs=pl.BlockSpec((_BR, D), lambda i: (i, 0)),
        out_shape=jax.ShapeDtypeStruct((L, D), jnp.float32),
    )(gathered, wie, bie.reshape(1, -1), woe, boe.reshape(1, -1),
      wil, bil.reshape(1, -1), wol, bol.reshape(1, -1))


def kernel(node_embeddings, atoms_edge, atoms_label,
           W_inner_edge, b_inner_edge, W_outer_edge, b_outer_edge,
           W_inner_label, b_inner_label, W_outer_label, b_outer_label):
    N, D = node_embeddings.shape
    E2 = atoms_edge.shape[0]
    NL = atoms_label.shape[0]
    L = E2 + NL

    info = plsc.get_sparse_core_info()
    NC, NS = info.num_cores, info.num_subcores
    NW = NC * NS
    align = NW * _CH
    n_chunks_total = -(-L // align)
    L_pad = n_chunks_total * align
    n_chunks_per_worker = L_pad // (NW * _CH)

    pad = L_pad - L
    idx_all = jnp.concatenate([
        atoms_edge, atoms_label,
        jnp.zeros((pad,), dtype=jnp.int32),
    ])

    gather_k = _make_sc_gather(L_pad, D, n_chunks_per_worker, NC, NS)
    gathered = gather_k(idx_all, node_embeddings)

    n_edge_blocks = E2 // _BR
    output_messages = _mlp_call(
        gathered, L, n_edge_blocks, D,
        W_inner_edge, b_inner_edge, W_outer_edge, b_outer_edge,
        W_inner_label, b_inner_label, W_outer_label, b_outer_label)
    output_indices = idx_all[:L]
    return (output_messages, output_indices)
